# Initial kernel scaffold; baseline (speedup 1.0000x reference)
import jax
import jax.numpy as jnp
from jax.experimental import pallas as pl


def kernel(x_miner, x_validator, x_full, x_storage, x_light, w_comp, root_comp, b_comp, w_val, root_val, b_val, w_srv, root_srv, b_srv, edge_index_competes, edge_type_competes, edge_index_validates, edge_index_serves):
    raise NotImplementedError("write your pallas kernel here")


# speedup vs baseline: 2.0469x; 2.0469x over previous
import functools

import jax
import jax.numpy as jnp
from jax import lax
from jax.experimental import pallas as pl
from jax.experimental.pallas import tpu as pltpu
from jax.experimental.pallas import tpu_sc as plsc

_N = 50000
_D = 128
_R = 5
_CH = 128
_NSUB = 16
_NCORE = 2

_KC = 152
_COMP_ROWS = _NSUB * _KC
_KV = 80
_VAL_ROWS = _NSUB * _KV

_BN = 1000
_GRID = _N // _BN
_STRIDE = _N
_ACC_ROWS = 250112
_ZC = _ACC_ROWS // _NSUB
_SV_ROWS = 50048
_ZV = _SV_ROWS // _NSUB
_PAD_C = _R * _N
_PAD_V = _N


def _m8(v):
    return pl.multiple_of(v, 8)


_SC_PARAMS = pltpu.CompilerParams(use_tc_tiling_on_sc=False)

_MESH = plsc.VectorSubcoreMesh(core_axis_name="c", subcore_axis_name="s",
                               num_cores=_NCORE, num_subcores=_NSUB)


@functools.partial(
    pl.kernel,
    out_type=(jax.ShapeDtypeStruct((_ACC_ROWS,), jnp.float32),
              jax.ShapeDtypeStruct((_SV_ROWS,), jnp.float32),
              jax.ShapeDtypeStruct((_SV_ROWS,), jnp.float32)),
    mesh=_MESH,
    compiler_params=_SC_PARAMS,
    scratch_types=(
        pltpu.VMEM((_KC, _CH), jnp.int32),
        pltpu.VMEM((_CH,), jnp.float32),
        pltpu.VMEM_SHARED((_ACC_ROWS,), jnp.float32),
        pltpu.VMEM_SHARED((_SV_ROWS,), jnp.float32),
        pltpu.VMEM_SHARED((_SV_ROWS,), jnp.float32),
    ),
)
def _sc_counts(idx_c, idx_v, idx_s, ones_in, z1,
               cnt_c, cnt_v, cnt_s,
               idx_b, ones_v, acc_c, acc_v, acc_s):
    ci = lax.axis_index("c")
    si = lax.axis_index("s")
    pltpu.sync_copy(ones_in, ones_v)

    @pl.when(ci == 0)
    def _():
        pltpu.sync_copy(z1, acc_c.at[pl.ds(_m8(si * _ZC), _ZC)])

    @pl.when(ci == 1)
    def _():
        pltpu.sync_copy(z1.at[pl.ds(0, _ZV)], acc_v.at[pl.ds(_m8(si * _ZV), _ZV)])
        pltpu.sync_copy(z1.at[pl.ds(0, _ZV)], acc_s.at[pl.ds(_m8(si * _ZV), _ZV)])

    plsc.subcore_barrier()

    @pl.when(ci == 0)
    def _():
        pltpu.sync_copy(idx_c.at[pl.ds(_m8(si * _KC), _KC)], idx_b)

        def body(j, carry):
            pltpu.sync_copy(ones_v, acc_c.at[idx_b.at[j]], add=True)
            return carry
        lax.fori_loop(0, _KC, body, 0)

    @pl.when(ci == 1)
    def _():
        for idx_in, acc in ((idx_v, acc_v), (idx_s, acc_s)):
            pltpu.sync_copy(idx_in.at[pl.ds(_m8(si * _KV), _KV)],
                            idx_b.at[pl.ds(0, _KV)])

            def body(j, carry):
                pltpu.sync_copy(ones_v, acc.at[idx_b.at[j]], add=True)
                return carry
            lax.fori_loop(0, _KV, body, 0)

    plsc.subcore_barrier()

    @pl.when(ci == 0)
    def _():
        pltpu.sync_copy(acc_c.at[pl.ds(_m8(si * _ZC), _ZC)],
                        cnt_c.at[pl.ds(_m8(si * _ZC), _ZC)])

    @pl.when(ci == 1)
    def _():
        pltpu.sync_copy(acc_v.at[pl.ds(_m8(si * _ZV), _ZV)],
                        cnt_v.at[pl.ds(_m8(si * _ZV), _ZV)])
        pltpu.sync_copy(acc_s.at[pl.ds(_m8(si * _ZV), _ZV)],
                        cnt_s.at[pl.ds(_m8(si * _ZV), _ZV)])


@functools.partial(
    pl.kernel,
    out_type=(jax.ShapeDtypeStruct((4, _SV_ROWS, 32), jnp.float32),
              jax.ShapeDtypeStruct((4, _SV_ROWS, 32), jnp.float32)),
    mesh=_MESH,
    compiler_params=_SC_PARAMS,
    scratch_types=(
        pltpu.VMEM((_CH,), jnp.int32),
        pltpu.VMEM((_CH,), jnp.int32),
        pltpu.VMEM((_CH, 32), jnp.float32),
        pltpu.VMEM_SHARED((_SV_ROWS, 32), jnp.float32),
        pltpu.SemaphoreType.DMA,
    ),
)
def _sc_valsrv(xv, xs, src_v_in, dst_v_in, src_s_in, dst_s_in, z32,
               out_v, out_s,
               src_b, dst_b, rows_b, acc, sem):
    ci = lax.axis_index("c")
    si = lax.axis_index("s")

    def run(x, src_in, dst_in, out):
        for slab in range(4):
            pltpu.sync_copy(z32, acc.at[pl.ds(_m8(si * _ZV), _ZV)])
            plsc.subcore_barrier()

            def body(j, carry):
                pltpu.sync_copy(src_in.at[si * _KV + j], src_b)
                pltpu.async_copy(x.at[slab].at[src_b], rows_b, sem).wait()
                pltpu.sync_copy(dst_in.at[si * _KV + j], dst_b)
                pltpu.sync_copy(rows_b, acc.at[dst_b], add=True)
                return carry
            lax.fori_loop(0, _KV, body, 0)
            plsc.subcore_barrier()
            pltpu.sync_copy(acc.at[pl.ds(_m8(si * _ZV), _ZV)],
                            out.at[slab].at[pl.ds(_m8(si * _ZV), _ZV)])

    @pl.when(ci == 0)
    def _():
        run(xv, src_v_in, dst_v_in, out_v)

    @pl.when(ci == 1)
    def _():
        run(xs, src_s_in, dst_s_in, out_s)


@functools.partial(
    pl.kernel,
    out_type=jax.ShapeDtypeStruct((16, _ACC_ROWS, 8), jnp.float32),
    mesh=_MESH,
    compiler_params=_SC_PARAMS,
    scratch_types=(
        pltpu.VMEM((_CH,), jnp.int32),
        pltpu.VMEM((_CH,), jnp.int32),
        pltpu.VMEM((_CH, 8), jnp.float32),
        pltpu.VMEM_SHARED((_ACC_ROWS, 8), jnp.float32),
        pltpu.SemaphoreType.DMA,
    ),
)
def _sc_comp(x, src_in, idx_in, z8,
             s_out,
             src_b, idx_b, rows_b, acc, sem):
    ci = lax.axis_index("c")
    si = lax.axis_index("s")
    for k in range(8):
        slab = ci * 8 + k
        pltpu.sync_copy(z8, acc.at[pl.ds(_m8(si * _ZC), _ZC)])
        plsc.subcore_barrier()

        def body(j, carry):
            pltpu.sync_copy(src_in.at[si * _KC + j], src_b)
            pltpu.async_copy(x.at[slab].at[src_b], rows_b, sem).wait()
            pltpu.sync_copy(idx_in.at[si * _KC + j], idx_b)
            pltpu.sync_copy(rows_b, acc.at[idx_b], add=True)
            return carry
        lax.fori_loop(0, _KC, body, 0)
        plsc.subcore_barrier()
        pltpu.sync_copy(acc.at[pl.ds(_m8(si * _ZC), _ZC)],
                        s_out.at[slab].at[pl.ds(_m8(si * _ZC), _ZC)])


def _lrelu(v):
    return jnp.where(v > 0, v, 0.01 * v)


def _tc_comp_body(x_ref, s0, s1, s2, s3, s4, c0, c1, c2, c3, c4,
                  root_ref, w_ref, b_ref, o_ref):
    acc = jnp.dot(x_ref[...], root_ref[...],
                  preferred_element_type=jnp.float32) + b_ref[...]
    for r, (s_ref, c_ref) in enumerate(
            zip((s0, s1, s2, s3, s4), (c0, c1, c2, c3, c4))):
        inv = 1.0 / jnp.maximum(c_ref[0, 0, :], 1.0)
        acc = acc + jnp.dot(s_ref[...] * inv[:, None], w_ref[r],
                            preferred_element_type=jnp.float32)
    o_ref[...] = _lrelu(acc)


def _combine_comp(x, s, cnt5, root, w, b):
    in_specs = [pl.BlockSpec((_BN, _D), lambda i: (i, 0))]
    for r in range(_R):
        in_specs.append(pl.BlockSpec((_BN, _D), lambda i, r=r: (r * 50 + i, 0)))
    for r in range(_R):
        in_specs.append(
            pl.BlockSpec((1, 1, _BN), lambda i, r=r: (r * 50 + i, 0, 0)))
    in_specs += [pl.BlockSpec((_D, _D), lambda i: (0, 0)),
                 pl.BlockSpec((_R, _D, _D), lambda i: (0, 0, 0)),
                 pl.BlockSpec((1, _D), lambda i: (0, 0))]
    return pl.pallas_call(
        _tc_comp_body,
        grid=(_GRID,),
        in_specs=in_specs,
        out_specs=pl.BlockSpec((_BN, _D), lambda i: (i, 0)),
        out_shape=jax.ShapeDtypeStruct((_N, _D), jnp.float32),
    )(x, s, s, s, s, s, cnt5, cnt5, cnt5, cnt5, cnt5, root, w, b)


def _tc_single_body(x_ref, s_ref, c_ref, root_ref, w_ref, b_ref, o_ref):
    inv = 1.0 / jnp.maximum(c_ref[0, 0, :], 1.0)
    acc = jnp.dot(x_ref[...], root_ref[...],
                  preferred_element_type=jnp.float32) + b_ref[...]
    acc = acc + jnp.dot(s_ref[...] * inv[:, None], w_ref[...],
                        preferred_element_type=jnp.float32)
    o_ref[...] = _lrelu(acc)


def _combine_single(x, s, cnt, root, w, b):
    return pl.pallas_call(
        _tc_single_body,
        grid=(_GRID,),
        in_specs=[pl.BlockSpec((_BN, _D), lambda i: (i, 0)),
                  pl.BlockSpec((_BN, _D), lambda i: (i, 0)),
                  pl.BlockSpec((1, 1, _BN), lambda i: (i, 0, 0)),
                  pl.BlockSpec((_D, _D), lambda i: (0, 0)),
                  pl.BlockSpec((_D, _D), lambda i: (0, 0)),
                  pl.BlockSpec((1, _D), lambda i: (0, 0))],
        out_specs=pl.BlockSpec((_BN, _D), lambda i: (i, 0)),
        out_shape=jax.ShapeDtypeStruct((_N, _D), jnp.float32),
    )(x, s, cnt, root, w, b)


def _prep(a, rows, fill):
    pad = rows * _CH - a.shape[0]
    return jnp.concatenate(
        [a.astype(jnp.int32), jnp.full((pad,), fill, jnp.int32)]
    ).reshape(rows, _CH)


def kernel(x_miner, x_validator, x_full, x_storage, x_light,
           w_comp, root_comp, b_comp,
           w_val, root_val, b_val,
           w_srv, root_srv, b_srv,
           edge_index_competes, edge_type_competes,
           edge_index_validates, edge_index_serves):
    f32 = jnp.float32
    sc_, dc = edge_index_competes[0], edge_index_competes[1]
    sv, dv = edge_index_validates[0], edge_index_validates[1]
    ssrc, sdst = edge_index_serves[0], edge_index_serves[1]

    idx_cp = _prep(edge_type_competes * _STRIDE + dc, _COMP_ROWS, _PAD_C)
    src_cp = _prep(sc_, _COMP_ROWS, 0)
    src_vp = _prep(sv, _VAL_ROWS, 0)
    dst_vp = _prep(dv, _VAL_ROWS, _PAD_V)
    src_sp = _prep(ssrc, _VAL_ROWS, 0)
    dst_sp = _prep(sdst, _VAL_ROWS, _PAD_V)

    z1 = jnp.zeros((_ZC,), f32)
    z8 = jnp.zeros((_ZC, 8), f32)
    z32 = jnp.zeros((_ZV, 32), f32)
    ones = jnp.ones((_CH,), f32)

    cnt_c, cnt_v, cnt_s = _sc_counts(idx_cp, dst_vp, dst_sp, ones, z1)
    cnt_c5 = cnt_c[:_R * _N].reshape(_R * 50, 1, _BN)
    cnt_v2 = cnt_v[:_N].reshape(50, 1, _BN)
    cnt_s2 = cnt_s[:_N].reshape(50, 1, _BN)

    xv4 = x_validator.reshape(_N, 4, 32).transpose(1, 0, 2)
    xs4 = x_storage.reshape(_N, 4, 32).transpose(1, 0, 2)
    s_val3, s_srv3 = _sc_valsrv(xv4, xs4,
                                src_vp, dst_vp, src_sp, dst_sp, z32)
    s_val = s_val3.transpose(1, 0, 2).reshape(_SV_ROWS, _D)
    s_srv = s_srv3.transpose(1, 0, 2).reshape(_SV_ROWS, _D)

    xm, xf, xl = x_miner, x_full, x_light
    for l in range(3):
        xm3 = xm.reshape(_N, 16, 8).transpose(1, 0, 2)
        s_c3 = _sc_comp(xm3, src_cp, idx_cp, z8)
        s_c = s_c3.transpose(1, 0, 2).reshape(_ACC_ROWS, _D)
        xm = _combine_comp(xm, s_c, cnt_c5, root_comp[l], w_comp[l],
                           b_comp[l].reshape(1, _D))
        xf = _combine_single(xf, s_val, cnt_v2, root_val[l], w_val[l, 0],
                             b_val[l].reshape(1, _D))
        xl = _combine_single(xl, s_srv, cnt_s2, root_srv[l], w_srv[l, 0],
                             b_srv[l].reshape(1, _D))
    return jnp.stack([xm, xf, xl])


# speedup vs baseline: 2.5679x; 1.2546x over previous
import functools

import jax
import jax.numpy as jnp
from jax import lax
from jax.experimental import pallas as pl
from jax.experimental.pallas import tpu as pltpu
from jax.experimental.pallas import tpu_sc as plsc

_N = 50000
_D = 128
_R = 5
_CH = 128
_NSUB = 16
_NCORE = 2

_KC = 152
_COMP_ROWS = _NSUB * _KC
_KV = 80
_VAL_ROWS = _NSUB * _KV

_BN = 1000
_GRID = _N // _BN
_STRIDE = _N
_ACC_ROWS = 250112
_ZC = _ACC_ROWS // _NSUB
_SV_ROWS = 50048
_ZV = _SV_ROWS // _NSUB
_PAD_C = _R * _N
_PAD_V = _N


def _m8(v):
    return pl.multiple_of(v, 8)


_SC_PARAMS = pltpu.CompilerParams(use_tc_tiling_on_sc=False)

_MESH = plsc.VectorSubcoreMesh(core_axis_name="c", subcore_axis_name="s",
                               num_cores=_NCORE, num_subcores=_NSUB)


@functools.partial(
    pl.kernel,
    out_type=(jax.ShapeDtypeStruct((_ACC_ROWS,), jnp.float32),
              jax.ShapeDtypeStruct((_SV_ROWS,), jnp.float32),
              jax.ShapeDtypeStruct((_SV_ROWS,), jnp.float32)),
    mesh=_MESH,
    compiler_params=_SC_PARAMS,
    scratch_types=(
        pltpu.VMEM((_KC, _CH), jnp.int32),
        pltpu.VMEM((_CH,), jnp.float32),
        pltpu.VMEM_SHARED((_ACC_ROWS,), jnp.float32),
        pltpu.VMEM_SHARED((_SV_ROWS,), jnp.float32),
        pltpu.VMEM_SHARED((_SV_ROWS,), jnp.float32),
    ),
)
def _sc_counts(idx_c, idx_v, idx_s, ones_in, z1,
               cnt_c, cnt_v, cnt_s,
               idx_b, ones_v, acc_c, acc_v, acc_s):
    ci = lax.axis_index("c")
    si = lax.axis_index("s")
    pltpu.sync_copy(ones_in, ones_v)

    @pl.when(ci == 0)
    def _():
        pltpu.sync_copy(z1, acc_c.at[pl.ds(_m8(si * _ZC), _ZC)])

    @pl.when(ci == 1)
    def _():
        pltpu.sync_copy(z1.at[pl.ds(0, _ZV)], acc_v.at[pl.ds(_m8(si * _ZV), _ZV)])
        pltpu.sync_copy(z1.at[pl.ds(0, _ZV)], acc_s.at[pl.ds(_m8(si * _ZV), _ZV)])

    plsc.subcore_barrier()

    @pl.when(ci == 0)
    def _():
        pltpu.sync_copy(idx_c.at[pl.ds(_m8(si * _KC), _KC)], idx_b)

        def body(j, carry):
            pltpu.sync_copy(ones_v, acc_c.at[idx_b.at[j]], add=True)
            return carry
        lax.fori_loop(0, _KC, body, 0)

    @pl.when(ci == 1)
    def _():
        for idx_in, acc in ((idx_v, acc_v), (idx_s, acc_s)):
            pltpu.sync_copy(idx_in.at[pl.ds(_m8(si * _KV), _KV)],
                            idx_b.at[pl.ds(0, _KV)])

            def body(j, carry):
                pltpu.sync_copy(ones_v, acc.at[idx_b.at[j]], add=True)
                return carry
            lax.fori_loop(0, _KV, body, 0)

    plsc.subcore_barrier()

    @pl.when(ci == 0)
    def _():
        pltpu.sync_copy(acc_c.at[pl.ds(_m8(si * _ZC), _ZC)],
                        cnt_c.at[pl.ds(_m8(si * _ZC), _ZC)])

    @pl.when(ci == 1)
    def _():
        pltpu.sync_copy(acc_v.at[pl.ds(_m8(si * _ZV), _ZV)],
                        cnt_v.at[pl.ds(_m8(si * _ZV), _ZV)])
        pltpu.sync_copy(acc_s.at[pl.ds(_m8(si * _ZV), _ZV)],
                        cnt_s.at[pl.ds(_m8(si * _ZV), _ZV)])


@functools.partial(
    pl.kernel,
    out_type=(jax.ShapeDtypeStruct((_SV_ROWS, _D), jnp.float32),
              jax.ShapeDtypeStruct((_SV_ROWS, _D), jnp.float32)),
    mesh=_MESH,
    compiler_params=_SC_PARAMS,
    scratch_types=(
        pltpu.VMEM((_CH,), jnp.int32),
        pltpu.VMEM((_CH,), jnp.int32),
        pltpu.VMEM((_CH, 32), jnp.float32),
        pltpu.VMEM_SHARED((_SV_ROWS, 32), jnp.float32),
        pltpu.SemaphoreType.DMA,
    ),
)
def _sc_valsrv(xv, xs, src_v_in, dst_v_in, src_s_in, dst_s_in, z32,
               out_v, out_s,
               src_b, dst_b, rows_b, acc, sem):
    ci = lax.axis_index("c")
    si = lax.axis_index("s")

    def run(x, src_in, dst_in, out):
        for slab in range(4):
            pltpu.sync_copy(z32, acc.at[pl.ds(_m8(si * _ZV), _ZV)])
            plsc.subcore_barrier()

            def body(j, carry):
                pltpu.sync_copy(src_in.at[si * _KV + j], src_b)
                pltpu.async_copy(x.at[slab].at[src_b], rows_b, sem).wait()
                pltpu.sync_copy(dst_in.at[si * _KV + j], dst_b)
                pltpu.sync_copy(rows_b, acc.at[dst_b], add=True)
                return carry
            lax.fori_loop(0, _KV, body, 0)
            plsc.subcore_barrier()
            pltpu.sync_copy(acc.at[pl.ds(_m8(si * _ZV), _ZV)],
                            out.at[pl.ds(_m8(si * _ZV), _ZV),
                                   pl.ds(_m8(slab * 32), 32)])

    @pl.when(ci == 0)
    def _():
        run(xv, src_v_in, dst_v_in, out_v)

    @pl.when(ci == 1)
    def _():
        run(xs, src_s_in, dst_s_in, out_s)


@functools.partial(
    pl.kernel,
    out_type=jax.ShapeDtypeStruct((_ACC_ROWS, _D), jnp.float32),
    mesh=_MESH,
    compiler_params=_SC_PARAMS,
    scratch_types=(
        pltpu.VMEM((_CH,), jnp.int32),
        pltpu.VMEM((_CH,), jnp.int32),
        pltpu.VMEM((_CH, 8), jnp.float32),
        pltpu.VMEM_SHARED((_ACC_ROWS, 8), jnp.float32),
        pltpu.SemaphoreType.DMA,
    ),
)
def _sc_comp(x, src_in, idx_in, z8,
             s_out,
             src_b, idx_b, rows_b, acc, sem):
    ci = lax.axis_index("c")
    si = lax.axis_index("s")
    for k in range(8):
        slab = ci * 8 + k
        pltpu.sync_copy(z8, acc.at[pl.ds(_m8(si * _ZC), _ZC)])
        plsc.subcore_barrier()

        def body(j, carry):
            pltpu.sync_copy(src_in.at[si * _KC + j], src_b)
            pltpu.async_copy(x.at[slab].at[src_b], rows_b, sem).wait()
            pltpu.sync_copy(idx_in.at[si * _KC + j], idx_b)
            pltpu.sync_copy(rows_b, acc.at[idx_b], add=True)
            return carry
        lax.fori_loop(0, _KC, body, 0)
        plsc.subcore_barrier()
        pltpu.sync_copy(acc.at[pl.ds(_m8(si * _ZC), _ZC)],
                        s_out.at[pl.ds(_m8(si * _ZC), _ZC),
                                 pl.ds(_m8(slab * 8), 8)])


def _lrelu(v):
    return jnp.where(v > 0, v, 0.01 * v)


def _tc_comp_body(x_ref, s0, s1, s2, s3, s4, c0, c1, c2, c3, c4,
                  root_ref, w_ref, b_ref, o_ref):
    acc = jnp.dot(x_ref[...], root_ref[...],
                  preferred_element_type=jnp.float32) + b_ref[...]
    for r, (s_ref, c_ref) in enumerate(
            zip((s0, s1, s2, s3, s4), (c0, c1, c2, c3, c4))):
        inv = 1.0 / jnp.maximum(c_ref[0, 0, :], 1.0)
        acc = acc + jnp.dot(s_ref[...] * inv[:, None], w_ref[r],
                            preferred_element_type=jnp.float32)
    o_ref[...] = _lrelu(acc)


def _combine_comp(x, s, cnt5, root, w, b):
    in_specs = [pl.BlockSpec((_BN, _D), lambda i: (i, 0))]
    for r in range(_R):
        in_specs.append(pl.BlockSpec((_BN, _D), lambda i, r=r: (r * 50 + i, 0)))
    for r in range(_R):
        in_specs.append(
            pl.BlockSpec((1, 1, _BN), lambda i, r=r: (r * 50 + i, 0, 0)))
    in_specs += [pl.BlockSpec((_D, _D), lambda i: (0, 0)),
                 pl.BlockSpec((_R, _D, _D), lambda i: (0, 0, 0)),
                 pl.BlockSpec((1, _D), lambda i: (0, 0))]
    return pl.pallas_call(
        _tc_comp_body,
        grid=(_GRID,),
        in_specs=in_specs,
        out_specs=pl.BlockSpec((_BN, _D), lambda i: (i, 0)),
        out_shape=jax.ShapeDtypeStruct((_N, _D), jnp.float32),
    )(x, s, s, s, s, s, cnt5, cnt5, cnt5, cnt5, cnt5, root, w, b)


def _tc_single_body(x_ref, s_ref, c_ref, root_ref, w_ref, b_ref, o_ref):
    inv = 1.0 / jnp.maximum(c_ref[0, 0, :], 1.0)
    acc = jnp.dot(x_ref[...], root_ref[...],
                  preferred_element_type=jnp.float32) + b_ref[...]
    acc = acc + jnp.dot(s_ref[...] * inv[:, None], w_ref[...],
                        preferred_element_type=jnp.float32)
    o_ref[...] = _lrelu(acc)


def _combine_single(x, s, cnt, root, w, b):
    return pl.pallas_call(
        _tc_single_body,
        grid=(_GRID,),
        in_specs=[pl.BlockSpec((_BN, _D), lambda i: (i, 0)),
                  pl.BlockSpec((_BN, _D), lambda i: (i, 0)),
                  pl.BlockSpec((1, 1, _BN), lambda i: (i, 0, 0)),
                  pl.BlockSpec((_D, _D), lambda i: (0, 0)),
                  pl.BlockSpec((_D, _D), lambda i: (0, 0)),
                  pl.BlockSpec((1, _D), lambda i: (0, 0))],
        out_specs=pl.BlockSpec((_BN, _D), lambda i: (i, 0)),
        out_shape=jax.ShapeDtypeStruct((_N, _D), jnp.float32),
    )(x, s, cnt, root, w, b)


def _prep(a, rows, fill):
    pad = rows * _CH - a.shape[0]
    return jnp.concatenate(
        [a.astype(jnp.int32), jnp.full((pad,), fill, jnp.int32)]
    ).reshape(rows, _CH)


def kernel(x_miner, x_validator, x_full, x_storage, x_light,
           w_comp, root_comp, b_comp,
           w_val, root_val, b_val,
           w_srv, root_srv, b_srv,
           edge_index_competes, edge_type_competes,
           edge_index_validates, edge_index_serves):
    f32 = jnp.float32
    sc_, dc = edge_index_competes[0], edge_index_competes[1]
    sv, dv = edge_index_validates[0], edge_index_validates[1]
    ssrc, sdst = edge_index_serves[0], edge_index_serves[1]

    idx_cp = _prep(edge_type_competes * _STRIDE + dc, _COMP_ROWS, _PAD_C)
    src_cp = _prep(sc_, _COMP_ROWS, 0)
    src_vp = _prep(sv, _VAL_ROWS, 0)
    dst_vp = _prep(dv, _VAL_ROWS, _PAD_V)
    src_sp = _prep(ssrc, _VAL_ROWS, 0)
    dst_sp = _prep(sdst, _VAL_ROWS, _PAD_V)

    z1 = jnp.zeros((_ZC,), f32)
    z8 = jnp.zeros((_ZC, 8), f32)
    z32 = jnp.zeros((_ZV, 32), f32)
    ones = jnp.ones((_CH,), f32)

    cnt_c, cnt_v, cnt_s = _sc_counts(idx_cp, dst_vp, dst_sp, ones, z1)
    cnt_c5 = cnt_c[:_R * _N].reshape(_R * 50, 1, _BN)
    cnt_v2 = cnt_v[:_N].reshape(50, 1, _BN)
    cnt_s2 = cnt_s[:_N].reshape(50, 1, _BN)

    xv4 = x_validator.reshape(_N, 4, 32).transpose(1, 0, 2)
    xs4 = x_storage.reshape(_N, 4, 32).transpose(1, 0, 2)
    s_val, s_srv = _sc_valsrv(xv4, xs4,
                              src_vp, dst_vp, src_sp, dst_sp, z32)

    xm, xf, xl = x_miner, x_full, x_light
    for l in range(3):
        xm3 = xm.reshape(_N, 16, 8).transpose(1, 0, 2)
        s_c = _sc_comp(xm3, src_cp, idx_cp, z8)
        xm = _combine_comp(xm, s_c, cnt_c5, root_comp[l], w_comp[l],
                           b_comp[l].reshape(1, _D))
        xf = _combine_single(xf, s_val, cnt_v2, root_val[l], w_val[l, 0],
                             b_val[l].reshape(1, _D))
        xl = _combine_single(xl, s_srv, cnt_s2, root_srv[l], w_srv[l, 0],
                             b_srv[l].reshape(1, _D))
    return jnp.stack([xm, xf, xl])


# speedup vs baseline: 4.1838x; 1.6292x over previous
import functools

import jax
import jax.numpy as jnp
from jax import lax
from jax.experimental import pallas as pl
from jax.experimental.pallas import tpu as pltpu
from jax.experimental.pallas import tpu_sc as plsc

_N = 50000
_D = 128
_R = 5
_CH = 128
_NSUB = 16
_NCORE = 2

_KC = 152
_COMP_ROWS = _NSUB * _KC
_KV = 80
_VAL_ROWS = _NSUB * _KV

_BN = 1000
_GRID = _N // _BN
_STRIDE = _N
_ACC_ROWS = 250112
_ZC = _ACC_ROWS // _NSUB
_SV_ROWS = 50048
_ZV = _SV_ROWS // _NSUB
_PAD_C = _R * _N
_PAD_V = _N


def _m8(v):
    return pl.multiple_of(v, 8)


_SC_PARAMS = pltpu.CompilerParams(use_tc_tiling_on_sc=False)

_MESH = plsc.VectorSubcoreMesh(core_axis_name="c", subcore_axis_name="s",
                               num_cores=_NCORE, num_subcores=_NSUB)


@functools.partial(
    pl.kernel,
    out_type=(jax.ShapeDtypeStruct((_ACC_ROWS,), jnp.float32),
              jax.ShapeDtypeStruct((_SV_ROWS,), jnp.float32),
              jax.ShapeDtypeStruct((_SV_ROWS,), jnp.float32)),
    mesh=_MESH,
    compiler_params=_SC_PARAMS,
    scratch_types=(
        pltpu.VMEM((_KC, _CH), jnp.int32),
        pltpu.VMEM((_CH,), jnp.float32),
        pltpu.VMEM_SHARED((_ACC_ROWS,), jnp.float32),
        pltpu.VMEM_SHARED((_SV_ROWS,), jnp.float32),
        pltpu.VMEM_SHARED((_SV_ROWS,), jnp.float32),
    ),
)
def _sc_counts(idx_c, idx_v, idx_s, ones_in, z1,
               cnt_c, cnt_v, cnt_s,
               idx_b, ones_v, acc_c, acc_v, acc_s):
    ci = lax.axis_index("c")
    si = lax.axis_index("s")
    pltpu.sync_copy(ones_in, ones_v)

    @pl.when(ci == 0)
    def _():
        pltpu.sync_copy(z1, acc_c.at[pl.ds(_m8(si * _ZC), _ZC)])

    @pl.when(ci == 1)
    def _():
        pltpu.sync_copy(z1.at[pl.ds(0, _ZV)], acc_v.at[pl.ds(_m8(si * _ZV), _ZV)])
        pltpu.sync_copy(z1.at[pl.ds(0, _ZV)], acc_s.at[pl.ds(_m8(si * _ZV), _ZV)])

    plsc.subcore_barrier()

    @pl.when(ci == 0)
    def _():
        pltpu.sync_copy(idx_c.at[pl.ds(_m8(si * _KC), _KC)], idx_b)

        def body(j, carry):
            pltpu.sync_copy(ones_v, acc_c.at[idx_b.at[j]], add=True)
            return carry
        lax.fori_loop(0, _KC, body, 0)

    @pl.when(ci == 1)
    def _():
        for idx_in, acc in ((idx_v, acc_v), (idx_s, acc_s)):
            pltpu.sync_copy(idx_in.at[pl.ds(_m8(si * _KV), _KV)],
                            idx_b.at[pl.ds(0, _KV)])

            def body(j, carry):
                pltpu.sync_copy(ones_v, acc.at[idx_b.at[j]], add=True)
                return carry
            lax.fori_loop(0, _KV, body, 0)

    plsc.subcore_barrier()

    @pl.when(ci == 0)
    def _():
        pltpu.sync_copy(acc_c.at[pl.ds(_m8(si * _ZC), _ZC)],
                        cnt_c.at[pl.ds(_m8(si * _ZC), _ZC)])

    @pl.when(ci == 1)
    def _():
        pltpu.sync_copy(acc_v.at[pl.ds(_m8(si * _ZV), _ZV)],
                        cnt_v.at[pl.ds(_m8(si * _ZV), _ZV)])
        pltpu.sync_copy(acc_s.at[pl.ds(_m8(si * _ZV), _ZV)],
                        cnt_s.at[pl.ds(_m8(si * _ZV), _ZV)])


def _edge_pass(x_slab, edges, acc, tile_chunk0, nblk, eblk, rows3, sems3):
    def body(blk, carry):
        base = _m8((tile_chunk0 + blk * 8) * 2)
        pltpu.sync_copy(edges.at[pl.ds(base, 16)], eblk)
        descs = {
            0: pltpu.async_copy(x_slab.at[eblk.at[0]], rows3[0], sems3[0]),
            1: pltpu.async_copy(x_slab.at[eblk.at[2]], rows3[1], sems3[1]),
        }
        for u in range(8):
            if u + 2 < 8:
                v = u + 2
                descs[v] = pltpu.async_copy(x_slab.at[eblk.at[2 * v]],
                                            rows3[v % 3], sems3[v % 3])
            descs[u].wait()
            pltpu.sync_copy(rows3[u % 3], acc.at[eblk.at[2 * u + 1]], add=True)
        return carry
    lax.fori_loop(0, nblk, body, 0)


@functools.partial(
    pl.kernel,
    out_type=(jax.ShapeDtypeStruct((_SV_ROWS, _D), jnp.float32),
              jax.ShapeDtypeStruct((_SV_ROWS, _D), jnp.float32)),
    mesh=_MESH,
    compiler_params=_SC_PARAMS,
    scratch_types=(
        pltpu.VMEM((16, _CH), jnp.int32),
        pltpu.VMEM((_CH, 32), jnp.float32),
        pltpu.VMEM((_CH, 32), jnp.float32),
        pltpu.VMEM((_CH, 32), jnp.float32),
        pltpu.VMEM_SHARED((_SV_ROWS, 32), jnp.float32),
        pltpu.SemaphoreType.DMA,
        pltpu.SemaphoreType.DMA,
        pltpu.SemaphoreType.DMA,
    ),
)
def _sc_valsrv(xv, xs, ed_v, ed_s, z32,
               out_v, out_s,
               eblk, r0, r1, r2, acc, s0, s1, s2):
    ci = lax.axis_index("c")
    si = lax.axis_index("s")

    def run(x, edges, out):
        for slab in range(4):
            pltpu.sync_copy(z32, acc.at[pl.ds(_m8(si * _ZV), _ZV)])
            plsc.subcore_barrier()
            _edge_pass(x.at[slab], edges, acc, si * _KV, _KV // 8,
                       eblk, (r0, r1, r2), (s0, s1, s2))
            plsc.subcore_barrier()
            pltpu.sync_copy(acc.at[pl.ds(_m8(si * _ZV), _ZV)],
                            out.at[pl.ds(_m8(si * _ZV), _ZV),
                                   pl.ds(_m8(slab * 32), 32)])

    @pl.when(ci == 0)
    def _():
        run(xv, ed_v, out_v)

    @pl.when(ci == 1)
    def _():
        run(xs, ed_s, out_s)


@functools.partial(
    pl.kernel,
    out_type=jax.ShapeDtypeStruct((_ACC_ROWS, _D), jnp.float32),
    mesh=_MESH,
    compiler_params=_SC_PARAMS,
    scratch_types=(
        pltpu.VMEM((16, _CH), jnp.int32),
        pltpu.VMEM((_CH, 8), jnp.float32),
        pltpu.VMEM((_CH, 8), jnp.float32),
        pltpu.VMEM((_CH, 8), jnp.float32),
        pltpu.VMEM_SHARED((_ACC_ROWS, 8), jnp.float32),
        pltpu.SemaphoreType.DMA,
        pltpu.SemaphoreType.DMA,
        pltpu.SemaphoreType.DMA,
    ),
)
def _sc_comp(x, ed_c, z8,
             s_out,
             eblk, r0, r1, r2, acc, s0, s1, s2):
    ci = lax.axis_index("c")
    si = lax.axis_index("s")
    for k in range(8):
        slab = ci * 8 + k
        pltpu.sync_copy(z8, acc.at[pl.ds(_m8(si * _ZC), _ZC)])
        plsc.subcore_barrier()
        _edge_pass(x.at[slab], ed_c, acc, si * _KC, _KC // 8,
                   eblk, (r0, r1, r2), (s0, s1, s2))
        plsc.subcore_barrier()
        pltpu.sync_copy(acc.at[pl.ds(_m8(si * _ZC), _ZC)],
                        s_out.at[pl.ds(_m8(si * _ZC), _ZC),
                                 pl.ds(_m8(slab * 8), 8)])


def _lrelu(v):
    return jnp.where(v > 0, v, 0.01 * v)


def _tc_comp_body(x_ref, s0, s1, s2, s3, s4, c0, c1, c2, c3, c4,
                  root_ref, w_ref, b_ref, o_ref):
    acc = jnp.dot(x_ref[...], root_ref[...],
                  preferred_element_type=jnp.float32) + b_ref[...]
    for r, (s_ref, c_ref) in enumerate(
            zip((s0, s1, s2, s3, s4), (c0, c1, c2, c3, c4))):
        inv = 1.0 / jnp.maximum(c_ref[0, 0, :], 1.0)
        acc = acc + jnp.dot(s_ref[...] * inv[:, None], w_ref[r],
                            preferred_element_type=jnp.float32)
    o_ref[...] = _lrelu(acc)


def _combine_comp(x, s, cnt5, root, w, b):
    in_specs = [pl.BlockSpec((_BN, _D), lambda i: (i, 0))]
    for r in range(_R):
        in_specs.append(pl.BlockSpec((_BN, _D), lambda i, r=r: (r * 50 + i, 0)))
    for r in range(_R):
        in_specs.append(
            pl.BlockSpec((1, 1, _BN), lambda i, r=r: (r * 50 + i, 0, 0)))
    in_specs += [pl.BlockSpec((_D, _D), lambda i: (0, 0)),
                 pl.BlockSpec((_R, _D, _D), lambda i: (0, 0, 0)),
                 pl.BlockSpec((1, _D), lambda i: (0, 0))]
    return pl.pallas_call(
        _tc_comp_body,
        grid=(_GRID,),
        in_specs=in_specs,
        out_specs=pl.BlockSpec((_BN, _D), lambda i: (i, 0)),
        out_shape=jax.ShapeDtypeStruct((_N, _D), jnp.float32),
    )(x, s, s, s, s, s, cnt5, cnt5, cnt5, cnt5, cnt5, root, w, b)


def _tc_single_body(x_ref, s_ref, c_ref, root_ref, w_ref, b_ref, o_ref):
    inv = 1.0 / jnp.maximum(c_ref[0, 0, :], 1.0)
    acc = jnp.dot(x_ref[...], root_ref[...],
                  preferred_element_type=jnp.float32) + b_ref[...]
    acc = acc + jnp.dot(s_ref[...] * inv[:, None], w_ref[...],
                        preferred_element_type=jnp.float32)
    o_ref[...] = _lrelu(acc)


def _combine_single(x, s, cnt, root, w, b):
    return pl.pallas_call(
        _tc_single_body,
        grid=(_GRID,),
        in_specs=[pl.BlockSpec((_BN, _D), lambda i: (i, 0)),
                  pl.BlockSpec((_BN, _D), lambda i: (i, 0)),
                  pl.BlockSpec((1, 1, _BN), lambda i: (i, 0, 0)),
                  pl.BlockSpec((_D, _D), lambda i: (0, 0)),
                  pl.BlockSpec((_D, _D), lambda i: (0, 0)),
                  pl.BlockSpec((1, _D), lambda i: (0, 0))],
        out_specs=pl.BlockSpec((_BN, _D), lambda i: (i, 0)),
        out_shape=jax.ShapeDtypeStruct((_N, _D), jnp.float32),
    )(x, s, cnt, root, w, b)


def _prep(a, rows, fill):
    pad = rows * _CH - a.shape[0]
    return jnp.concatenate(
        [a.astype(jnp.int32), jnp.full((pad,), fill, jnp.int32)]
    ).reshape(rows, _CH)


def kernel(x_miner, x_validator, x_full, x_storage, x_light,
           w_comp, root_comp, b_comp,
           w_val, root_val, b_val,
           w_srv, root_srv, b_srv,
           edge_index_competes, edge_type_competes,
           edge_index_validates, edge_index_serves):
    f32 = jnp.float32
    sc_, dc = edge_index_competes[0], edge_index_competes[1]
    sv, dv = edge_index_validates[0], edge_index_validates[1]
    ssrc, sdst = edge_index_serves[0], edge_index_serves[1]

    idx_cp = _prep(edge_type_competes * _STRIDE + dc, _COMP_ROWS, _PAD_C)
    src_cp = _prep(sc_, _COMP_ROWS, 0)
    src_vp = _prep(sv, _VAL_ROWS, 0)
    dst_vp = _prep(dv, _VAL_ROWS, _PAD_V)
    src_sp = _prep(ssrc, _VAL_ROWS, 0)
    dst_sp = _prep(sdst, _VAL_ROWS, _PAD_V)

    z1 = jnp.zeros((_ZC,), f32)
    z8 = jnp.zeros((_ZC, 8), f32)
    z32 = jnp.zeros((_ZV, 32), f32)
    ones = jnp.ones((_CH,), f32)

    cnt_c, cnt_v, cnt_s = _sc_counts(idx_cp, dst_vp, dst_sp, ones, z1)
    cnt_c5 = cnt_c[:_R * _N].reshape(_R * 50, 1, _BN)
    cnt_v2 = cnt_v[:_N].reshape(50, 1, _BN)
    cnt_s2 = cnt_s[:_N].reshape(50, 1, _BN)

    xv4 = x_validator.reshape(_N, 4, 32).transpose(1, 0, 2)
    xs4 = x_storage.reshape(_N, 4, 32).transpose(1, 0, 2)
    ed_v = jnp.stack([src_vp, dst_vp], axis=1).reshape(2 * _VAL_ROWS, _CH)
    ed_s = jnp.stack([src_sp, dst_sp], axis=1).reshape(2 * _VAL_ROWS, _CH)
    s_val, s_srv = _sc_valsrv(xv4, xs4, ed_v, ed_s, z32)

    ed_c = jnp.stack([src_cp, idx_cp], axis=1).reshape(2 * _COMP_ROWS, _CH)

    xm, xf, xl = x_miner, x_full, x_light
    for l in range(3):
        xm3 = xm.reshape(_N, 16, 8).transpose(1, 0, 2)
        s_c = _sc_comp(xm3, ed_c, z8)
        xm = _combine_comp(xm, s_c, cnt_c5, root_comp[l], w_comp[l],
                           b_comp[l].reshape(1, _D))
        xf = _combine_single(xf, s_val, cnt_v2, root_val[l], w_val[l, 0],
                             b_val[l].reshape(1, _D))
        xl = _combine_single(xl, s_srv, cnt_s2, root_srv[l], w_srv[l, 0],
                             b_srv[l].reshape(1, _D))
    return jnp.stack([xm, xf, xl])
